# baseline (speedup 1.0000x reference)
import jax
import jax.numpy as jnp
from jax import lax
from jax.experimental import pallas as pl

_E = 8
_T = 8192
_NEG_INF = float("-inf")


def _fused_body(x_ref, w_ref, b_ref, gates_ref, idx_ref):
    logits = lax.dot_general(
        w_ref[...], x_ref[...],
        (((1,), (1,)), ((), ())),
        preferred_element_type=jnp.float32,
    ) + b_ref[...]

    m1 = jnp.max(logits, axis=0, keepdims=True)
    i1 = jnp.full((1, _T), _E - 1, dtype=jnp.int32)
    for e in range(_E - 2, -1, -1):
        i1 = jnp.where(logits[e:e + 1, :] == m1, jnp.int32(e), i1)

    eiota = lax.broadcasted_iota(jnp.int32, (_E, _T), 0)
    masked = jnp.where(eiota == i1, _NEG_INF, logits)
    m2 = jnp.max(masked, axis=0, keepdims=True)
    i2 = jnp.full((1, _T), _E - 1, dtype=jnp.int32)
    for e in range(_E - 2, -1, -1):
        i2 = jnp.where(masked[e:e + 1, :] == m2, jnp.int32(e), i2)

    e2 = jnp.exp(m2 - m1)
    r = 1.0 / (1.0 + e2)
    g1 = r
    g2 = e2 * r

    gates_ref[0] = jnp.where(eiota == i1, g1, jnp.where(eiota == i2, g2, 0.0))
    idx_ref[0] = jnp.concatenate([i1, i2], axis=0)


def kernel(x, W, b):
    B, N, D = x.shape
    tokens = B * N
    grid = tokens // _T
    x2 = x.reshape(tokens, D)
    b2 = b.reshape(_E, 1)

    gates_t, idx_t = pl.pallas_call(
        _fused_body,
        grid=(grid,),
        in_specs=[
            pl.BlockSpec((_T, D), lambda i: (i, 0)),
            pl.BlockSpec((_E, D), lambda i: (0, 0)),
            pl.BlockSpec((_E, 1), lambda i: (0, 0)),
        ],
        out_specs=[
            pl.BlockSpec((1, _E, _T), lambda i: (i, 0, 0)),
            pl.BlockSpec((1, 2, _T), lambda i: (i, 0, 0)),
        ],
        out_shape=[
            jax.ShapeDtypeStruct((grid, _E, _T), jnp.float32),
            jax.ShapeDtypeStruct((grid, 2, _T), jnp.int32),
        ],
    )(x2, W, b2)

    full_gates = gates_t.transpose(0, 2, 1).reshape(B, N, _E)
    topk_idx = idx_t.transpose(0, 2, 1).reshape(B, N, 2)
    return (full_gates, topk_idx)


# speedup vs baseline: 1.0559x; 1.0559x over previous
import jax
import jax.numpy as jnp
from jax import lax
from jax.experimental import pallas as pl

_E = 8
_T = 4096
_NEG_INF = float("-inf")


def _fused_body(x_ref, w_ref, b_ref, gates_ref, idx_ref):
    logits = lax.dot_general(
        w_ref[...], x_ref[...],
        (((1,), (1,)), ((), ())),
        preferred_element_type=jnp.float32,
    ) + b_ref[...]

    m1 = jnp.max(logits, axis=0, keepdims=True)
    i1 = jnp.full((1, _T), _E - 1, dtype=jnp.int32)
    for e in range(_E - 2, -1, -1):
        i1 = jnp.where(logits[e:e + 1, :] == m1, jnp.int32(e), i1)

    eiota = lax.broadcasted_iota(jnp.int32, (_E, _T), 0)
    masked = jnp.where(eiota == i1, _NEG_INF, logits)
    m2 = jnp.max(masked, axis=0, keepdims=True)
    i2 = jnp.full((1, _T), _E - 1, dtype=jnp.int32)
    for e in range(_E - 2, -1, -1):
        i2 = jnp.where(masked[e:e + 1, :] == m2, jnp.int32(e), i2)

    e2 = jnp.exp(m2 - m1)
    r = 1.0 / (1.0 + e2)
    g1 = r
    g2 = e2 * r

    gates_ref[0] = jnp.where(eiota == i1, g1, jnp.where(eiota == i2, g2, 0.0))
    idx_ref[0] = jnp.concatenate([i1, i2], axis=0)


def kernel(x, W, b):
    B, N, D = x.shape
    tokens = B * N
    grid = tokens // _T
    x2 = x.reshape(tokens, D)
    b2 = b.reshape(_E, 1)

    gates_t, idx_t = pl.pallas_call(
        _fused_body,
        grid=(grid,),
        in_specs=[
            pl.BlockSpec((_T, D), lambda i: (i, 0)),
            pl.BlockSpec((_E, D), lambda i: (0, 0)),
            pl.BlockSpec((_E, 1), lambda i: (0, 0)),
        ],
        out_specs=[
            pl.BlockSpec((1, _E, _T), lambda i: (i, 0, 0)),
            pl.BlockSpec((1, 2, _T), lambda i: (i, 0, 0)),
        ],
        out_shape=[
            jax.ShapeDtypeStruct((grid, _E, _T), jnp.float32),
            jax.ShapeDtypeStruct((grid, 2, _T), jnp.int32),
        ],
    )(x2, W, b2)

    full_gates = gates_t.transpose(0, 2, 1).reshape(B, N, _E)
    topk_idx = idx_t.transpose(0, 2, 1).reshape(B, N, 2)
    return (full_gates, topk_idx)


# speedup vs baseline: 1.0777x; 1.0207x over previous
import jax
import jax.numpy as jnp
from jax import lax
from jax.experimental import pallas as pl

_E = 8
_T = 4096
_NEG_INF = float("-inf")


def _fused_body(xa_ref, xb_ref, w_ref, b_ref, gates_ref, idx_ref):
    la = lax.dot_general(
        w_ref[...], xa_ref[...],
        (((1,), (1,)), ((), ())),
        preferred_element_type=jnp.float32,
    )
    lb = lax.dot_general(
        w_ref[...], xb_ref[...],
        (((1,), (1,)), ((), ())),
        preferred_element_type=jnp.float32,
    )
    logits = jnp.concatenate([la, lb], axis=1) + b_ref[...]

    m1 = jnp.max(logits, axis=0, keepdims=True)
    i1 = jnp.full((1, _T), _E - 1, dtype=jnp.int32)
    for e in range(_E - 2, -1, -1):
        i1 = jnp.where(logits[e:e + 1, :] == m1, jnp.int32(e), i1)

    eiota = lax.broadcasted_iota(jnp.int32, (_E, _T), 0)
    masked = jnp.where(eiota == i1, _NEG_INF, logits)
    m2 = jnp.max(masked, axis=0, keepdims=True)
    i2 = jnp.full((1, _T), _E - 1, dtype=jnp.int32)
    for e in range(_E - 2, -1, -1):
        i2 = jnp.where(masked[e:e + 1, :] == m2, jnp.int32(e), i2)

    e2 = jnp.exp(m2 - m1)
    r = 1.0 / (1.0 + e2)
    g1 = r
    g2 = e2 * r

    gates_ref[0] = jnp.where(eiota == i1, g1, jnp.where(eiota == i2, g2, 0.0))
    idx_ref[0] = jnp.concatenate([i1, i2], axis=0)


def kernel(x, W, b):
    B, N, D = x.shape
    tokens = B * N
    grid = tokens // _T
    x2 = x.reshape(tokens, D)
    b2 = b.reshape(_E, 1)

    gates_t, idx_t = pl.pallas_call(
        _fused_body,
        grid=(grid,),
        in_specs=[
            pl.BlockSpec((_T // 2, D), lambda i: (2 * i, 0)),
            pl.BlockSpec((_T // 2, D), lambda i: (2 * i + 1, 0)),
            pl.BlockSpec((_E, D), lambda i: (0, 0)),
            pl.BlockSpec((_E, 1), lambda i: (0, 0)),
        ],
        out_specs=[
            pl.BlockSpec((1, _E, _T), lambda i: (i, 0, 0)),
            pl.BlockSpec((1, 2, _T), lambda i: (i, 0, 0)),
        ],
        out_shape=[
            jax.ShapeDtypeStruct((grid, _E, _T), jnp.float32),
            jax.ShapeDtypeStruct((grid, 2, _T), jnp.int32),
        ],
    )(x2, x2, W, b2)

    full_gates = gates_t.transpose(0, 2, 1).reshape(B, N, _E)
    topk_idx = idx_t.transpose(0, 2, 1).reshape(B, N, 2)
    return (full_gates, topk_idx)
